# baseline (speedup 1.0000x reference)
import functools

import jax
import jax.numpy as jnp
from jax import lax
from jax.experimental import pallas as pl
from jax.experimental.pallas import tpu as pltpu
from jax.experimental.pallas import tpu_sc as plsc

_BATCH = 4096
_SEQ = 50
_D = 64
_V = 100000

_NC = 2
_NS = 16
_NW = _NC * _NS
_FPW = _D // _NW
_GROUPS = _BATCH // 16


def _slab_kernel(idx_hbm, tab_hbm, out_hbm, rowbuf, idxbuf, stagebuf):
    wid = lax.axis_index("s") * _NC + lax.axis_index("c")

    def per_feature(fi, carry):
        f = wid * _FPW + fi
        pltpu.sync_copy(tab_hbm.at[f], rowbuf)

        def per_row(j, carry2):
            pltpu.sync_copy(idx_hbm.at[j], idxbuf)

            def per_group(g, carry3):
                idxv = idxbuf[pl.ds(g * 16, 16)]
                vals = plsc.load_gather(rowbuf, [idxv])
                stagebuf[pl.ds(g * 16, 16)] = vals
                return carry3

            lax.fori_loop(0, _GROUPS, per_group, 0, unroll=8)
            pltpu.sync_copy(stagebuf, out_hbm.at[j, f])
            return carry2

        lax.fori_loop(0, _SEQ, per_row, 0)
        return carry

    lax.fori_loop(0, _FPW, per_feature, 0)


@jax.jit
def _embed_gather(idx_t, tab_t):
    mesh = plsc.VectorSubcoreMesh(core_axis_name="c", subcore_axis_name="s")
    k = functools.partial(
        pl.kernel,
        mesh=mesh,
        out_type=jax.ShapeDtypeStruct((_SEQ, _D, _BATCH), jnp.float32),
        scratch_types=[
            pltpu.VMEM((_V,), jnp.float32),
            pltpu.VMEM((_BATCH,), jnp.int32),
            pltpu.VMEM((_BATCH,), jnp.float32),
        ],
        compiler_params=pltpu.CompilerParams(needs_layout_passes=False),
    )(_slab_kernel)
    return k(idx_t, tab_t)


def kernel(inputs, embedding):
    idx_t = inputs.astype(jnp.int32).T
    tab_t = jnp.asarray(embedding, jnp.float32).T
    p = _embed_gather(idx_t, tab_t)
    return p.transpose(2, 0, 1)


# speedup vs baseline: 2.6848x; 2.6848x over previous
import functools

import jax
import jax.numpy as jnp
from jax import lax
from jax.experimental import pallas as pl
from jax.experimental.pallas import tpu as pltpu
from jax.experimental.pallas import tpu_sc as plsc

_BATCH = 4096
_SEQ = 50
_D = 64
_V = 100000

_NC = 2
_NS = 16
_NW = _NC * _NS
_FPW = _D // _NW
_GROUPS = _BATCH // 16


def _slab_kernel(idx_hbm, tab_hbm, out_hbm, rowbuf, ibufs, sbufs, isems, wsems):
    wid = lax.axis_index("s") * _NC + lax.axis_index("c")

    def gather_row(ibuf, sbuf):
        @plsc.parallel_loop(0, _GROUPS, 1, unroll=8)
        def _(g):
            iv = ibuf[pl.ds(g * 16, 16)]
            sbuf[pl.ds(g * 16, 16)] = plsc.load_gather(rowbuf, [iv])

    for fi in range(_FPW):
        f = wid * _FPW + fi
        pltpu.sync_copy(tab_hbm.at[f], rowbuf)

        def j2body(j2, carry):
            j0 = 2 * j2
            j1 = j0 + 1
            h0 = pltpu.async_copy(idx_hbm.at[j0], ibufs[0], isems[0])
            h1 = pltpu.async_copy(idx_hbm.at[j1], ibufs[1], isems[1])
            h0.wait()
            gather_row(ibufs[0], sbufs[0])
            w0 = pltpu.async_copy(sbufs[0], out_hbm.at[j0, f], wsems[0])
            h1.wait()
            gather_row(ibufs[1], sbufs[1])
            w1 = pltpu.async_copy(sbufs[1], out_hbm.at[j1, f], wsems[1])
            w0.wait()
            w1.wait()
            return carry

        lax.fori_loop(0, _SEQ // 2, j2body, 0)


@jax.jit
def _embed_gather(idx_t, tab_t):
    mesh = plsc.VectorSubcoreMesh(core_axis_name="c", subcore_axis_name="s")
    k = functools.partial(
        pl.kernel,
        mesh=mesh,
        out_type=jax.ShapeDtypeStruct((_SEQ, _D, _BATCH), jnp.float32),
        scratch_types=[
            pltpu.VMEM((_V,), jnp.float32),
            [pltpu.VMEM((_BATCH,), jnp.int32) for _ in range(2)],
            [pltpu.VMEM((_BATCH,), jnp.float32) for _ in range(2)],
            [pltpu.SemaphoreType.DMA for _ in range(2)],
            [pltpu.SemaphoreType.DMA for _ in range(2)],
        ],
        compiler_params=pltpu.CompilerParams(needs_layout_passes=False),
    )(_slab_kernel)
    return k(idx_t, tab_t)


def kernel(inputs, embedding):
    idx_t = inputs.astype(jnp.int32).T
    tab_t = jnp.asarray(embedding, jnp.float32).T
    p = _embed_gather(idx_t, tab_t)
    return p.transpose(2, 0, 1)


# speedup vs baseline: 3.7116x; 1.3825x over previous
import functools

import jax
import jax.numpy as jnp
from jax import lax
from jax.experimental import pallas as pl
from jax.experimental.pallas import tpu as pltpu
from jax.experimental.pallas import tpu_sc as plsc

_BATCH = 4096
_SEQ = 50
_D = 64
_V = 100000

_NC = 2
_NS = 16
_NW = _NC * _NS
_FPW = _D // _NW
_GROUPS = _BATCH // 16
_NIB = 4
_NSB = 2
_JBODY = _SEQ // _NIB


def _slab_kernel(idx_hbm, tab_hbm, out_hbm, rowbuf, ibufs, sbufs, isems, wsems):
    wid = lax.axis_index("s") * _NC + lax.axis_index("c")

    def gather_row(ibuf, sbuf):
        @plsc.parallel_loop(0, _GROUPS, 1, unroll=8)
        def _(g):
            iv = ibuf[pl.ds(g * 16, 16)]
            sbuf[pl.ds(g * 16, 16)] = plsc.load_gather(rowbuf, [iv])

    def fire_idx(j, p):
        pltpu.async_copy(idx_hbm.at[j], ibufs[p], isems[p])

    def wait_idx(j, p):
        pltpu.make_async_copy(idx_hbm.at[j], ibufs[p], isems[p]).wait()

    def wait_write(j, f, s):
        pltpu.make_async_copy(sbufs[s], out_hbm.at[j, f], wsems[s]).wait()

    for fi in range(_FPW):
        f = wid * _FPW + fi
        pltpu.sync_copy(tab_hbm.at[f], rowbuf)
        for p in range(_NIB):
            fire_idx(p, p)

        def jbody(i, carry):
            for p in range(_NIB):
                j = _NIB * i + p
                s = p % _NSB
                if p < _NSB:
                    @pl.when(i > 0)
                    def _():
                        wait_write(j, f, s)
                else:
                    wait_write(j, f, s)
                wait_idx(j, p)
                gather_row(ibufs[p], sbufs[s])
                pltpu.async_copy(sbufs[s], out_hbm.at[j, f], wsems[s])
                nxt = j + _NIB
                @pl.when(nxt < _SEQ)
                def _():
                    fire_idx(nxt, p)
            return carry

        lax.fori_loop(0, _JBODY, jbody, 0)

        for p in range(_SEQ - _NIB * _JBODY):
            j = _NIB * _JBODY + p
            s = p % _NSB
            wait_write(j, f, s)
            wait_idx(j, p)
            gather_row(ibufs[p], sbufs[s])
            pltpu.async_copy(sbufs[s], out_hbm.at[j, f], wsems[s])
        for s in range(_NSB):
            wait_write(0, f, s)


@jax.jit
def _embed_gather(idx_t, tab_t):
    mesh = plsc.VectorSubcoreMesh(core_axis_name="c", subcore_axis_name="s")
    k = functools.partial(
        pl.kernel,
        mesh=mesh,
        out_type=jax.ShapeDtypeStruct((_SEQ, _D, _BATCH), jnp.float32),
        scratch_types=[
            pltpu.VMEM((_V,), jnp.float32),
            [pltpu.VMEM((_BATCH,), jnp.int32) for _ in range(_NIB)],
            [pltpu.VMEM((_BATCH,), jnp.float32) for _ in range(_NSB)],
            [pltpu.SemaphoreType.DMA for _ in range(_NIB)],
            [pltpu.SemaphoreType.DMA for _ in range(_NSB)],
        ],
        compiler_params=pltpu.CompilerParams(needs_layout_passes=False),
    )(_slab_kernel)
    return k(idx_t, tab_t)


def kernel(inputs, embedding):
    idx_t = inputs.astype(jnp.int32).T
    tab_t = jnp.asarray(embedding, jnp.float32).T
    p = _embed_gather(idx_t, tab_t)
    return p.transpose(2, 0, 1)
